# baseline (speedup 1.0000x reference)
import jax
import jax.numpy as jnp
from jax.experimental import pallas as pl
from jax.experimental.pallas import tpu as pltpu

_B, _M, _D = 16, 16, 768
_T, _E, _K = 3, 8, 2
_EH, _BH, _TH = 1024, 768, 512
_COEF = 0.01
_N = _B * _M
_HI = jax.lax.Precision.HIGHEST
_BF = jnp.bfloat16


def _cv2(v):
    mean = jnp.sum(v) / _E
    var = jnp.sum((v - mean) ** 2) / (_E - 1)
    return var / (mean * mean + 1e-10)


def _bdot(a, b):
    return jnp.dot(a.astype(_BF), b.astype(_BF), preferred_element_type=jnp.float32)


def _moe_loss_kernel(x_ref, scores_ref, fc1w_ref, fc1b_ref, fc2w_ref, fc2b_ref,
                     wg_ref, ew1_ref, eb1_ref, ew2_ref, eb2_ref,
                     tw1_ref, tb1_ref, tw2_ref, tb2_ref,
                     out_ref,
                     preds_bf, gi, gv, aux, y, tw1_s, wb1, wb2, sem):
    e = pl.program_id(0)
    slot = jax.lax.rem(e, 2)
    nxt = jax.lax.rem(e + 1, 2)

    _EHH = _EH // 2

    def _start(ee, sl):
        pltpu.make_async_copy(ew1_ref.at[ee, :, 0:_EHH], wb1.at[sl, :, 0:_EHH],
                              sem.at[sl, 0]).start()
        pltpu.make_async_copy(ew1_ref.at[ee, :, _EHH:_EH], wb1.at[sl, :, _EHH:_EH],
                              sem.at[sl, 1]).start()
        pltpu.make_async_copy(ew2_ref.at[ee, 0:_EHH], wb2.at[sl, 0:_EHH],
                              sem.at[sl, 2]).start()
        pltpu.make_async_copy(ew2_ref.at[ee, _EHH:_EH], wb2.at[sl, _EHH:_EH],
                              sem.at[sl, 3]).start()

    def _wait(ee, sl):
        pltpu.make_async_copy(ew1_ref.at[ee, :, 0:_EHH], wb1.at[sl, :, 0:_EHH],
                              sem.at[sl, 0]).wait()
        pltpu.make_async_copy(ew1_ref.at[ee, :, _EHH:_EH], wb1.at[sl, :, _EHH:_EH],
                              sem.at[sl, 1]).wait()
        pltpu.make_async_copy(ew2_ref.at[ee, 0:_EHH], wb2.at[sl, 0:_EHH],
                              sem.at[sl, 2]).wait()
        pltpu.make_async_copy(ew2_ref.at[ee, _EHH:_EH], wb2.at[sl, _EHH:_EH],
                              sem.at[sl, 3]).wait()

    @pl.when(e == 0)
    def _start_first():
        _start(0, 0)

    @pl.when(e == 0)
    def _prologue():
        x = x_ref[...]
        h1 = jnp.maximum(_bdot(x, fc1w_ref[...]) + fc1b_ref[...], 0.0)
        p = _bdot(h1, fc2w_ref[...]) + fc2b_ref[...]
        preds_bf[...] = p.astype(_BF)
        logits = jnp.dot(p, wg_ref[...], precision=_HI)
        ids = jax.lax.broadcasted_iota(jnp.int32, (_N, _E), 1)
        for j in range(_T):
            lg = logits[:, j * _E:(j + 1) * _E]
            v1 = jnp.max(lg, axis=1, keepdims=True)
            i1 = jnp.min(jnp.where(lg == v1, ids, _E), axis=1, keepdims=True)
            masked = jnp.where(ids == i1, -jnp.inf, lg)
            v2 = jnp.max(masked, axis=1, keepdims=True)
            i2 = jnp.min(jnp.where(masked == v2, ids, _E), axis=1, keepdims=True)
            t = jnp.exp(v2 - v1)
            den = 1.0 + t
            g1 = 1.0 / den
            g2 = t / den
            gates = jnp.where(ids == i1, g1, 0.0) + jnp.where(ids == i2, g2, 0.0)
            gi[j] = jnp.concatenate([i1, i2], axis=1)
            gv[j] = jnp.concatenate([g1, g2], axis=1)
            imp = jnp.sum(gates, axis=0, keepdims=True)
            load = jnp.sum((gates > 0.0).astype(jnp.float32), axis=0, keepdims=True)
            aux[j] = _cv2(imp) + _cv2(load)
        y[...] = jnp.zeros((_T, _N, _D), _BF)

    @pl.when(e < _T)
    def _stash_tower():
        tw1_s[e] = tw1_ref[0]

    @pl.when(e < _E - 1)
    def _start_next():
        _start(e + 1, nxt)

    _wait(e, slot)

    p = preds_bf[...]
    h = jnp.maximum(_bdot(p, wb1[slot]) + eb1_ref[0], 0.0)
    eo = (_bdot(h, wb2[slot]) + eb2_ref[0]).astype(_BF)
    for j in range(_T):
        ge = (jnp.where(gi[j, :, 0:1] == e, gv[j, :, 0:1], 0.0)
              + jnp.where(gi[j, :, 1:2] == e, gv[j, :, 1:2], 0.0)).astype(_BF)
        y[j] += ge * eo

    @pl.when(e == _E - 1)
    def _epilogue():
        n_r = jax.lax.broadcasted_iota(jnp.int32, (_N, _M), 0)
        m_r = jax.lax.broadcasted_iota(jnp.int32, (_N, _M), 1)
        r_sel = (n_r % _M == m_r).astype(jnp.float32)
        b_l = jax.lax.broadcasted_iota(jnp.int32, (_B, _N), 0)
        n_l = jax.lax.broadcasted_iota(jnp.int32, (_B, _N), 1)
        l_sel = (n_l // _M == b_l).astype(jnp.float32)
        ts = []
        lfs = []
        for j in range(_T):
            yh = jnp.maximum(_bdot(y[j], tw1_s[j]) + tb1_ref[j], 0.0)
            ts.append(_bdot(yh, tw2_ref[:, j:j + 1]) + tb2_ref[:, j:j + 1])
            sc = scores_ref[j]
            lab = (sc == jnp.max(sc, axis=1, keepdims=True)).astype(jnp.float32)
            lfs.append(jnp.sum(jnp.dot(l_sel.T, lab, precision=_HI) * r_sel,
                               axis=1, keepdims=True))
        t_all = jnp.concatenate(ts, axis=1)
        lab_all = jnp.concatenate(lfs, axis=1)
        bce = (jnp.maximum(t_all, 0.0) - t_all * lab_all
               + jnp.log1p(jnp.exp(-jnp.abs(t_all))))
        loss = jnp.sum(bce) / _M + _COEF * (aux[0] + aux[1] + aux[2])
        out_ref[...] = jnp.reshape(loss, (1, 1))


@jax.jit
def kernel(source_cls_embed, candidate_cls_embed, text_and_summaries_mask, scores,
           fc1_w, fc1_b, fc2_w, fc2_b, w_gate,
           expert_w1, expert_b1, expert_w2, expert_b2,
           tower_w1, tower_b1, tower_w2, tower_b2):
    x = candidate_cls_embed.reshape(_N, _D)
    scores_t = jnp.transpose(scores, (1, 0, 2))
    wg = jnp.transpose(w_gate, (1, 0, 2)).reshape(_D, _T * _E)
    tw2c = jnp.transpose(tower_w2[:, :, 0])
    tb2c = tower_b2[:, 0][None]
    out = pl.pallas_call(
        _moe_loss_kernel,
        grid=(_E,),
        in_specs=[
            pl.BlockSpec((_N, _D), lambda e: (0, 0)),
            pl.BlockSpec((_T, _B, _M), lambda e: (0, 0, 0)),
            pl.BlockSpec((_D, _BH), lambda e: (0, 0)),
            pl.BlockSpec((1, _BH), lambda e: (0, 0)),
            pl.BlockSpec((_BH, _D), lambda e: (0, 0)),
            pl.BlockSpec((1, _D), lambda e: (0, 0)),
            pl.BlockSpec((_D, _T * _E), lambda e: (0, 0)),
            pl.BlockSpec(memory_space=pltpu.MemorySpace.HBM),
            pl.BlockSpec((1, 1, _EH), lambda e: (e, 0, 0)),
            pl.BlockSpec(memory_space=pltpu.MemorySpace.HBM),
            pl.BlockSpec((1, 1, _D), lambda e: (e, 0, 0)),
            pl.BlockSpec((1, _D, _TH), lambda e: (jnp.minimum(e, _T - 1), 0, 0)),
            pl.BlockSpec((_T, _TH), lambda e: (0, 0)),
            pl.BlockSpec((_TH, _T), lambda e: (0, 0)),
            pl.BlockSpec((1, _T), lambda e: (0, 0)),
        ],
        out_specs=pl.BlockSpec((1, 1), lambda e: (0, 0)),
        out_shape=jax.ShapeDtypeStruct((1, 1), jnp.float32),
        scratch_shapes=[
            pltpu.VMEM((_N, _D), _BF),
            pltpu.VMEM((_T, _N, 2), jnp.int32),
            pltpu.VMEM((_T, _N, 2), jnp.float32),
            pltpu.SMEM((_T,), jnp.float32),
            pltpu.VMEM((_T, _N, _D), _BF),
            pltpu.VMEM((_T, _D, _TH), jnp.float32),
            pltpu.VMEM((2, _D, _EH), jnp.float32),
            pltpu.VMEM((2, _EH, _D), jnp.float32),
            pltpu.SemaphoreType.DMA((2, 4)),
        ],
        compiler_params=pltpu.CompilerParams(dimension_semantics=("arbitrary",)),
    )(x, scores_t, fc1_w, fc1_b[None], fc2_w, fc2_b[None], wg,
      expert_w1, expert_b1[:, None, :], expert_w2, expert_b2[:, None, :],
      tower_w1, tower_b1, tw2c, tb2c)
    return out[0, 0]


# speedup vs baseline: 1.0138x; 1.0138x over previous
import jax
import jax.numpy as jnp
from jax.experimental import pallas as pl
from jax.experimental.pallas import tpu as pltpu

_B, _M, _D = 16, 16, 768
_T, _E, _K = 3, 8, 2
_EH, _BH, _TH = 1024, 768, 512
_COEF = 0.01
_N = _B * _M
_HI = jax.lax.Precision.HIGHEST
_BF = jnp.bfloat16


def _cv2(v):
    mean = jnp.sum(v) / _E
    var = jnp.sum((v - mean) ** 2) / (_E - 1)
    return var / (mean * mean + 1e-10)


def _bdot(a, b):
    return jnp.dot(a.astype(_BF), b.astype(_BF), preferred_element_type=jnp.float32)


def _moe_loss_kernel(x_ref, scores_ref, fc1w_ref, fc1b_ref, fc2w_ref, fc2b_ref,
                     wg_ref, ew1_ref, eb1_ref, ew2_ref, eb2_ref,
                     tw1_ref, tb1_ref, tw2_ref, tb2_ref,
                     out_ref,
                     preds_bf, gi, gv, aux, y, tw1_s, wb1, wb2, sem):
    e = pl.program_id(0)
    slot = jax.lax.rem(e, 2)
    nxt = jax.lax.rem(e + 1, 2)

    _EHH = _EH // 2

    def _start(ee, sl):
        pltpu.make_async_copy(ew1_ref.at[ee, :, 0:_EHH], wb1.at[sl, :, 0:_EHH],
                              sem.at[sl, 0]).start()
        pltpu.make_async_copy(ew1_ref.at[ee, :, _EHH:_EH], wb1.at[sl, :, _EHH:_EH],
                              sem.at[sl, 1]).start()
        pltpu.make_async_copy(ew2_ref.at[ee, 0:_EHH], wb2.at[sl, 0:_EHH],
                              sem.at[sl, 2]).start()
        pltpu.make_async_copy(ew2_ref.at[ee, _EHH:_EH], wb2.at[sl, _EHH:_EH],
                              sem.at[sl, 3]).start()

    def _wait_piece(ee, sl, k):
        if k == 0:
            pltpu.make_async_copy(ew1_ref.at[ee, :, 0:_EHH], wb1.at[sl, :, 0:_EHH],
                                  sem.at[sl, 0]).wait()
        elif k == 1:
            pltpu.make_async_copy(ew1_ref.at[ee, :, _EHH:_EH], wb1.at[sl, :, _EHH:_EH],
                                  sem.at[sl, 1]).wait()
        elif k == 2:
            pltpu.make_async_copy(ew2_ref.at[ee, 0:_EHH], wb2.at[sl, 0:_EHH],
                                  sem.at[sl, 2]).wait()
        else:
            pltpu.make_async_copy(ew2_ref.at[ee, _EHH:_EH], wb2.at[sl, _EHH:_EH],
                                  sem.at[sl, 3]).wait()

    @pl.when(e == 0)
    def _start_first():
        _start(0, 0)

    @pl.when(e == 0)
    def _prologue():
        x = x_ref[...]
        h1 = jnp.maximum(_bdot(x, fc1w_ref[...]) + fc1b_ref[...], 0.0)
        p = _bdot(h1, fc2w_ref[...]) + fc2b_ref[...]
        preds_bf[...] = p.astype(_BF)
        logits = jnp.dot(p, wg_ref[...], precision=_HI)
        ids = jax.lax.broadcasted_iota(jnp.int32, (_N, _E), 1)
        for j in range(_T):
            lg = logits[:, j * _E:(j + 1) * _E]
            v1 = jnp.max(lg, axis=1, keepdims=True)
            i1 = jnp.min(jnp.where(lg == v1, ids, _E), axis=1, keepdims=True)
            masked = jnp.where(ids == i1, -jnp.inf, lg)
            v2 = jnp.max(masked, axis=1, keepdims=True)
            i2 = jnp.min(jnp.where(masked == v2, ids, _E), axis=1, keepdims=True)
            t = jnp.exp(v2 - v1)
            den = 1.0 + t
            g1 = 1.0 / den
            g2 = t / den
            gates = jnp.where(ids == i1, g1, 0.0) + jnp.where(ids == i2, g2, 0.0)
            gi[j] = jnp.concatenate([i1, i2], axis=1)
            gv[j] = jnp.concatenate([g1, g2], axis=1)
            imp = jnp.sum(gates, axis=0, keepdims=True)
            load = jnp.sum((gates > 0.0).astype(jnp.float32), axis=0, keepdims=True)
            aux[j] = _cv2(imp) + _cv2(load)
        y[...] = jnp.zeros((_T, _N, _D), _BF)

    @pl.when(e < _T)
    def _stash_tower():
        tw1_s[e] = tw1_ref[0]

    @pl.when(e < _E - 1)
    def _start_next():
        _start(e + 1, nxt)

    p = preds_bf[...]
    _wait_piece(e, slot, 0)
    ha = jnp.maximum(_bdot(p, wb1[slot][:, 0:_EHH]) + eb1_ref[0][:, 0:_EHH], 0.0)
    _wait_piece(e, slot, 1)
    hb = jnp.maximum(_bdot(p, wb1[slot][:, _EHH:_EH]) + eb1_ref[0][:, _EHH:_EH], 0.0)
    _wait_piece(e, slot, 2)
    eo1 = _bdot(ha, wb2[slot][0:_EHH])
    _wait_piece(e, slot, 3)
    eo = (eo1 + _bdot(hb, wb2[slot][_EHH:_EH]) + eb2_ref[0]).astype(_BF)
    for j in range(_T):
        ge = (jnp.where(gi[j, :, 0:1] == e, gv[j, :, 0:1], 0.0)
              + jnp.where(gi[j, :, 1:2] == e, gv[j, :, 1:2], 0.0)).astype(_BF)
        y[j] += ge * eo

    @pl.when(e == _E - 1)
    def _epilogue():
        n_r = jax.lax.broadcasted_iota(jnp.int32, (_N, _M), 0)
        m_r = jax.lax.broadcasted_iota(jnp.int32, (_N, _M), 1)
        r_sel = (n_r % _M == m_r).astype(jnp.float32)
        b_l = jax.lax.broadcasted_iota(jnp.int32, (_B, _N), 0)
        n_l = jax.lax.broadcasted_iota(jnp.int32, (_B, _N), 1)
        l_sel = (n_l // _M == b_l).astype(jnp.float32)
        ts = []
        lfs = []
        for j in range(_T):
            yh = jnp.maximum(_bdot(y[j], tw1_s[j]) + tb1_ref[j], 0.0)
            ts.append(_bdot(yh, tw2_ref[:, j:j + 1]) + tb2_ref[:, j:j + 1])
            sc = scores_ref[j]
            lab = (sc == jnp.max(sc, axis=1, keepdims=True)).astype(jnp.float32)
            lfs.append(jnp.sum(jnp.dot(l_sel.T, lab, precision=_HI) * r_sel,
                               axis=1, keepdims=True))
        t_all = jnp.concatenate(ts, axis=1)
        lab_all = jnp.concatenate(lfs, axis=1)
        bce = (jnp.maximum(t_all, 0.0) - t_all * lab_all
               + jnp.log1p(jnp.exp(-jnp.abs(t_all))))
        loss = jnp.sum(bce) / _M + _COEF * (aux[0] + aux[1] + aux[2])
        out_ref[...] = jnp.reshape(loss, (1, 1))


@jax.jit
def kernel(source_cls_embed, candidate_cls_embed, text_and_summaries_mask, scores,
           fc1_w, fc1_b, fc2_w, fc2_b, w_gate,
           expert_w1, expert_b1, expert_w2, expert_b2,
           tower_w1, tower_b1, tower_w2, tower_b2):
    x = candidate_cls_embed.reshape(_N, _D)
    scores_t = jnp.transpose(scores, (1, 0, 2))
    wg = jnp.transpose(w_gate, (1, 0, 2)).reshape(_D, _T * _E)
    tw2c = jnp.transpose(tower_w2[:, :, 0])
    tb2c = tower_b2[:, 0][None]
    out = pl.pallas_call(
        _moe_loss_kernel,
        grid=(_E,),
        in_specs=[
            pl.BlockSpec((_N, _D), lambda e: (0, 0)),
            pl.BlockSpec((_T, _B, _M), lambda e: (0, 0, 0)),
            pl.BlockSpec((_D, _BH), lambda e: (0, 0)),
            pl.BlockSpec((1, _BH), lambda e: (0, 0)),
            pl.BlockSpec((_BH, _D), lambda e: (0, 0)),
            pl.BlockSpec((1, _D), lambda e: (0, 0)),
            pl.BlockSpec((_D, _T * _E), lambda e: (0, 0)),
            pl.BlockSpec(memory_space=pltpu.MemorySpace.HBM),
            pl.BlockSpec((1, 1, _EH), lambda e: (e, 0, 0)),
            pl.BlockSpec(memory_space=pltpu.MemorySpace.HBM),
            pl.BlockSpec((1, 1, _D), lambda e: (e, 0, 0)),
            pl.BlockSpec((1, _D, _TH), lambda e: (jnp.minimum(e, _T - 1), 0, 0)),
            pl.BlockSpec((_T, _TH), lambda e: (0, 0)),
            pl.BlockSpec((_TH, _T), lambda e: (0, 0)),
            pl.BlockSpec((1, _T), lambda e: (0, 0)),
        ],
        out_specs=pl.BlockSpec((1, 1), lambda e: (0, 0)),
        out_shape=jax.ShapeDtypeStruct((1, 1), jnp.float32),
        scratch_shapes=[
            pltpu.VMEM((_N, _D), _BF),
            pltpu.VMEM((_T, _N, 2), jnp.int32),
            pltpu.VMEM((_T, _N, 2), jnp.float32),
            pltpu.SMEM((_T,), jnp.float32),
            pltpu.VMEM((_T, _N, _D), _BF),
            pltpu.VMEM((_T, _D, _TH), jnp.float32),
            pltpu.VMEM((2, _D, _EH), jnp.float32),
            pltpu.VMEM((2, _EH, _D), jnp.float32),
            pltpu.SemaphoreType.DMA((2, 4)),
        ],
        compiler_params=pltpu.CompilerParams(dimension_semantics=("arbitrary",)),
    )(x, scores_t, fc1_w, fc1_b[None], fc2_w, fc2_b[None], wg,
      expert_w1, expert_b1[:, None, :], expert_w2, expert_b2[:, None, :],
      tower_w1, tower_b1, tw2c, tb2c)
    return out[0, 0]
